# Initial kernel scaffold; baseline (speedup 1.0000x reference)
import functools

import jax
import jax.numpy as jnp
from jax import lax
from jax.experimental import pallas as pl
from jax.experimental.pallas import tpu as pltpu
from jax.experimental.pallas import tpu_sc as plsc

N = 10000
D = 128
NC = 2
NS = 16
NPAD = 10240
ROWS_PER_TILE = NPAD // NS
E = 320000
K = 128
EPT = 10240
EPAD = NC * NS * EPT
CHUNKS = EPT // K
RB = 1000
GRID = N // RB


def _make_agg(with_deg):
    mesh = plsc.VectorSubcoreMesh(core_axis_name="c", subcore_axis_name="s")
    out_type = [jax.ShapeDtypeStruct((NC, NPAD, D), jnp.float32)]
    scratch = [
        pltpu.VMEM((K,), jnp.int32),
        pltpu.VMEM((K,), jnp.int32),
        pltpu.VMEM((K, D), jnp.float32),
        pltpu.VMEM_SHARED((NPAD, D), jnp.float32),
        pltpu.SemaphoreType.DMA,
    ]
    if with_deg:
        out_type.append(jax.ShapeDtypeStruct((NC, NPAD, 16), jnp.float32))
        scratch += [
            pltpu.VMEM((K, 16), jnp.float32),
            pltpu.VMEM_SHARED((NPAD, 16), jnp.float32),
        ]

    def body(*refs):
        if with_deg:
            (p_hbm, src_hbm, dst_hbm, zrows_hbm, zdeg_hbm, ones_hbm,
             agg_out, deg_out, si_v, di_v, rows_v, acc_sh, sem,
             ones_v, deg_sh) = refs
        else:
            (p_hbm, src_hbm, dst_hbm, zrows_hbm,
             agg_out, si_v, di_v, rows_v, acc_sh, sem) = refs
        c = lax.axis_index("c")
        s = lax.axis_index("s")
        r0 = s * ROWS_PER_TILE
        pltpu.sync_copy(zrows_hbm, acc_sh.at[pl.ds(r0, ROWS_PER_TILE)])
        if with_deg:
            pltpu.sync_copy(zdeg_hbm, deg_sh.at[pl.ds(r0, ROWS_PER_TILE)])
            pltpu.sync_copy(ones_hbm, ones_v)
        plsc.subcore_barrier()
        ebase = (c * NS + s) * EPT

        def chunk(i, carry):
            base = ebase + i * K
            pltpu.sync_copy(src_hbm.at[pl.ds(base, K)], si_v)
            pltpu.sync_copy(dst_hbm.at[pl.ds(base, K)], di_v)
            pltpu.async_copy(p_hbm.at[si_v], rows_v, sem).wait()
            pltpu.sync_copy(rows_v, acc_sh.at[di_v], add=True)
            if with_deg:
                pltpu.sync_copy(ones_v, deg_sh.at[di_v], add=True)
            return carry

        lax.fori_loop(0, CHUNKS, chunk, 0)
        plsc.subcore_barrier()
        pltpu.sync_copy(acc_sh.at[pl.ds(r0, ROWS_PER_TILE)],
                        agg_out.at[c].at[pl.ds(r0, ROWS_PER_TILE)])
        if with_deg:
            pltpu.sync_copy(deg_sh.at[pl.ds(r0, ROWS_PER_TILE)],
                            deg_out.at[c].at[pl.ds(r0, ROWS_PER_TILE)])

    return pl.kernel(body, mesh=mesh, out_type=out_type, scratch_types=scratch)


_agg_deg = _make_agg(True)
_agg_only = _make_agg(False)


def _pre_body(x_ref, w_ref, p_ref, s_ref):
    h = jnp.dot(x_ref[...], w_ref[...], preferred_element_type=jnp.float32)
    p_ref[...] = h[:, :D]
    s_ref[...] = h[:, D:]


_pre = pl.pallas_call(
    _pre_body,
    grid=(GRID,),
    in_specs=[pl.BlockSpec((RB, D), lambda i: (i, 0)),
              pl.BlockSpec((D, 2 * D), lambda i: (0, 0))],
    out_specs=[pl.BlockSpec((RB, D), lambda i: (i, 0))] * 2,
    out_shape=[jax.ShapeDtypeStruct((N, D), jnp.float32)] * 2,
)


def _mid_body(s1_ref, a0_ref, a1_ref, d0_ref, d1_ref, b1_ref, w_ref,
              p_ref, s_ref):
    deg = d0_ref[:, 0:1] + d1_ref[:, 0:1]
    hn = (a0_ref[...] + a1_ref[...]) / jnp.maximum(deg, 1.0)
    h = jax.nn.sigmoid(s1_ref[...] + hn + b1_ref[...])
    hw = jnp.dot(h, w_ref[...], preferred_element_type=jnp.float32)
    p_ref[...] = hw[:, :D]
    s_ref[...] = hw[:, D:]


_mid = pl.pallas_call(
    _mid_body,
    grid=(GRID,),
    in_specs=[pl.BlockSpec((RB, D), lambda i: (i, 0)),
              pl.BlockSpec((RB, D), lambda i: (i, 0)),
              pl.BlockSpec((RB, D), lambda i: (i, 0)),
              pl.BlockSpec((RB, 16), lambda i: (i, 0)),
              pl.BlockSpec((RB, 16), lambda i: (i, 0)),
              pl.BlockSpec((1, D), lambda i: (0, 0)),
              pl.BlockSpec((D, 2 * D), lambda i: (0, 0))],
    out_specs=[pl.BlockSpec((RB, D), lambda i: (i, 0))] * 2,
    out_shape=[jax.ShapeDtypeStruct((N, D), jnp.float32)] * 2,
)


def _post_body(s2_ref, a0_ref, a1_ref, d0_ref, d1_ref, b2_ref, waw_ref,
               baw_ref, wt1_ref, bt1_ref, wt2_ref, bt2_ref, out_ref,
               hsum_ref, hmax_ref):
    i = pl.program_id(0)
    deg = d0_ref[:, 0:1] + d1_ref[:, 0:1]
    hn = (a0_ref[...] + a1_ref[...]) / jnp.maximum(deg, 1.0)
    h = jax.nn.sigmoid(s2_ref[...] + hn + b2_ref[...])
    w = jax.nn.sigmoid(
        jnp.dot(h, waw_ref[...], preferred_element_type=jnp.float32)[:, 0:1]
        + baw_ref[0, 0])
    psum = jnp.sum(w * h, axis=0, keepdims=True)
    pmax = jnp.max(h, axis=0, keepdims=True)

    @pl.when(i == 0)
    def _():
        hsum_ref[...] = psum
        hmax_ref[...] = pmax

    @pl.when(i > 0)
    def _():
        hsum_ref[...] = hsum_ref[...] + psum
        hmax_ref[...] = jnp.maximum(hmax_ref[...], pmax)

    @pl.when(i == GRID - 1)
    def _():
        g = jnp.concatenate([hsum_ref[...], hmax_ref[...]], axis=1)
        y1 = jax.nn.sigmoid(
            jnp.dot(g, wt1_ref[...], preferred_element_type=jnp.float32)
            + bt1_ref[...])
        y2 = jax.nn.sigmoid(
            jnp.dot(y1, wt2_ref[...], preferred_element_type=jnp.float32)[:, 0:1]
            + bt2_ref[0, 0])
        out_ref[...] = jnp.broadcast_to(y2, (1, D))


_post = pl.pallas_call(
    _post_body,
    grid=(GRID,),
    in_specs=[pl.BlockSpec((RB, D), lambda i: (i, 0)),
              pl.BlockSpec((RB, D), lambda i: (i, 0)),
              pl.BlockSpec((RB, D), lambda i: (i, 0)),
              pl.BlockSpec((RB, 16), lambda i: (i, 0)),
              pl.BlockSpec((RB, 16), lambda i: (i, 0)),
              pl.BlockSpec((1, D), lambda i: (0, 0)),
              pl.BlockSpec((D, D), lambda i: (0, 0)),
              pl.BlockSpec((1, D), lambda i: (0, 0)),
              pl.BlockSpec((2 * D, D), lambda i: (0, 0)),
              pl.BlockSpec((1, D), lambda i: (0, 0)),
              pl.BlockSpec((D, D), lambda i: (0, 0)),
              pl.BlockSpec((1, D), lambda i: (0, 0))],
    out_specs=pl.BlockSpec((1, D), lambda i: (0, 0)),
    out_shape=jax.ShapeDtypeStruct((1, D), jnp.float32),
    scratch_shapes=[pltpu.VMEM((1, D), jnp.float32),
                    pltpu.VMEM((1, D), jnp.float32)],
)


def kernel(n, edge_index, e, W_self1, W_neigh1, b1, W_self2, W_neigh2, b2,
           W_aw, b_aw, W_t1, b_t1, W_t2, b_t2):
    del e
    src = edge_index[0]
    dst = edge_index[1]
    pad = EPAD - E
    src_p = jnp.concatenate([src, jnp.zeros((pad,), jnp.int32)])
    dst_p = jnp.concatenate([dst, jnp.full((pad,), N, jnp.int32)])
    zrows = jnp.zeros((ROWS_PER_TILE, D), jnp.float32)
    zdeg = jnp.zeros((ROWS_PER_TILE, 16), jnp.float32)
    ones = jnp.ones((K, 16), jnp.float32)
    Wcat1 = jnp.concatenate([W_neigh1, W_self1], axis=1)
    Wcat2 = jnp.concatenate([W_neigh2, W_self2], axis=1)

    P1, S1 = _pre(n, Wcat1)
    agg1, deg = _agg_deg(P1, src_p, dst_p, zrows, zdeg, ones)
    P2, S2 = _mid(S1, agg1[0], agg1[1], deg[0], deg[1],
                  b1.reshape(1, D), Wcat2)
    agg2 = _agg_only(P2, src_p, dst_p, zrows)
    y = _post(S2, agg2[0], agg2[1], deg[0], deg[1], b2.reshape(1, D),
              jnp.pad(W_aw, ((0, 0), (0, D - 1))),
              jnp.broadcast_to(b_aw.reshape(1, 1), (1, D)),
              W_t1, b_t1.reshape(1, D),
              jnp.pad(W_t2, ((0, 0), (0, D - 1))),
              jnp.broadcast_to(b_t2.reshape(1, 1), (1, D)))
    return y[:, 0:1]


# speedup vs baseline: 4.5957x; 4.5957x over previous
import jax
import jax.numpy as jnp
from jax import lax
from jax.experimental import pallas as pl
from jax.experimental.pallas import tpu as pltpu
from jax.experimental.pallas import tpu_sc as plsc

N = 10000
D = 128
NC = 2
NS = 16
NPAD = 10240
ROWS_PER_TILE = NPAD // NS
E = 320000
K = 128
EPT = 10240
EPAD = NC * NS * EPT
CHUNKS = EPT // K
RB = 1000
GRID = N // RB

_MESH = plsc.VectorSubcoreMesh(core_axis_name="c", subcore_axis_name="s")


def _agg_body(p_hbm, src_hbm, dst_hbm, zrows_hbm,
              agg_out, si_v, di_v, rows_v, acc_sh, sem):
    c = lax.axis_index("c")
    s = lax.axis_index("s")
    r0 = s * ROWS_PER_TILE
    pltpu.sync_copy(zrows_hbm, rows_v)
    for j in range(ROWS_PER_TILE // K):
        pltpu.sync_copy(rows_v, acc_sh.at[pl.ds(r0 + j * K, K)])
    plsc.subcore_barrier()
    ebase = (c * NS + s) * EPT

    def chunk(i, carry):
        base = ebase + i * K
        pltpu.sync_copy(src_hbm.at[pl.ds(base, K)], si_v)
        pltpu.sync_copy(dst_hbm.at[pl.ds(base, K)], di_v)
        pltpu.async_copy(p_hbm.at[si_v], rows_v, sem).wait()
        pltpu.sync_copy(rows_v, acc_sh.at[di_v], add=True)
        return carry

    lax.fori_loop(0, CHUNKS, chunk, 0)
    plsc.subcore_barrier()
    obase = c * NPAD + r0
    for j in range(ROWS_PER_TILE // K):
        pltpu.sync_copy(acc_sh.at[pl.ds(r0 + j * K, K)], rows_v)
        pltpu.sync_copy(rows_v, agg_out.at[pl.ds(obase + j * K, K)])


_agg = pl.kernel(
    _agg_body,
    mesh=_MESH,
    out_type=[jax.ShapeDtypeStruct((NC * NPAD, D), jnp.float32)],
    scratch_types=[
        pltpu.VMEM((K,), jnp.int32),
        pltpu.VMEM((K,), jnp.int32),
        pltpu.VMEM((K, D), jnp.float32),
        pltpu.VMEM_SHARED((NPAD, D), jnp.float32),
        pltpu.SemaphoreType.DMA,
    ],
)


def _pre_body(x_ref, w_ref, p_ref, s_ref):
    h = jnp.dot(x_ref[...], w_ref[...], preferred_element_type=jnp.float32)
    p_ref[...] = h[:, :D]
    s_ref[...] = h[:, D:]


_pre = pl.pallas_call(
    _pre_body,
    grid=(GRID,),
    in_specs=[pl.BlockSpec((RB, D), lambda i: (i, 0)),
              pl.BlockSpec((D, 2 * D), lambda i: (0, 0))],
    out_specs=[pl.BlockSpec((RB, D), lambda i: (i, 0))] * 2,
    out_shape=[jax.ShapeDtypeStruct((N, D), jnp.float32)] * 2,
)


def _mid_body(s1_ref, a0_ref, a1_ref, d0_ref, d1_ref, b1_ref, w_ref,
              p_ref, s_ref):
    deg = d0_ref[:, 0:1] + d1_ref[:, 0:1]
    hn = (a0_ref[...] + a1_ref[...]) / jnp.maximum(deg, 1.0)
    h = jax.nn.sigmoid(s1_ref[...] + hn + b1_ref[...])
    hw = jnp.dot(h, w_ref[...], preferred_element_type=jnp.float32)
    p_ref[...] = hw[:, :D]
    s_ref[...] = hw[:, D:]


_mid = pl.pallas_call(
    _mid_body,
    grid=(GRID,),
    in_specs=[pl.BlockSpec((RB, D), lambda i: (i, 0)),
              pl.BlockSpec((RB, D), lambda i: (i, 0)),
              pl.BlockSpec((RB, D), lambda i: (i, 0)),
              pl.BlockSpec((RB, D), lambda i: (i, 0)),
              pl.BlockSpec((RB, D), lambda i: (i, 0)),
              pl.BlockSpec((1, D), lambda i: (0, 0)),
              pl.BlockSpec((D, 2 * D), lambda i: (0, 0))],
    out_specs=[pl.BlockSpec((RB, D), lambda i: (i, 0))] * 2,
    out_shape=[jax.ShapeDtypeStruct((N, D), jnp.float32)] * 2,
)


def _post_body(s2_ref, a0_ref, a1_ref, d0_ref, d1_ref, b2_ref, waw_ref,
               baw_ref, wt1_ref, bt1_ref, wt2_ref, bt2_ref, out_ref,
               hsum_ref, hmax_ref):
    i = pl.program_id(0)
    deg = d0_ref[:, 0:1] + d1_ref[:, 0:1]
    hn = (a0_ref[...] + a1_ref[...]) / jnp.maximum(deg, 1.0)
    h = jax.nn.sigmoid(s2_ref[...] + hn + b2_ref[...])
    w = jax.nn.sigmoid(
        jnp.dot(h, waw_ref[...], preferred_element_type=jnp.float32)[:, 0:1]
        + baw_ref[0, 0])
    psum = jnp.sum(w * h, axis=0, keepdims=True)
    pmax = jnp.max(h, axis=0, keepdims=True)

    @pl.when(i == 0)
    def _():
        hsum_ref[...] = psum
        hmax_ref[...] = pmax

    @pl.when(i > 0)
    def _():
        hsum_ref[...] = hsum_ref[...] + psum
        hmax_ref[...] = jnp.maximum(hmax_ref[...], pmax)

    @pl.when(i == GRID - 1)
    def _():
        g = jnp.concatenate([hsum_ref[...], hmax_ref[...]], axis=1)
        y1 = jax.nn.sigmoid(
            jnp.dot(g, wt1_ref[...], preferred_element_type=jnp.float32)
            + bt1_ref[...])
        y2 = jax.nn.sigmoid(
            jnp.dot(y1, wt2_ref[...], preferred_element_type=jnp.float32)[:, 0:1]
            + bt2_ref[0, 0])
        out_ref[...] = jnp.broadcast_to(y2, (1, D))


_post = pl.pallas_call(
    _post_body,
    grid=(GRID,),
    in_specs=[pl.BlockSpec((RB, D), lambda i: (i, 0)),
              pl.BlockSpec((RB, D), lambda i: (i, 0)),
              pl.BlockSpec((RB, D), lambda i: (i, 0)),
              pl.BlockSpec((RB, D), lambda i: (i, 0)),
              pl.BlockSpec((RB, D), lambda i: (i, 0)),
              pl.BlockSpec((1, D), lambda i: (0, 0)),
              pl.BlockSpec((D, D), lambda i: (0, 0)),
              pl.BlockSpec((1, D), lambda i: (0, 0)),
              pl.BlockSpec((2 * D, D), lambda i: (0, 0)),
              pl.BlockSpec((1, D), lambda i: (0, 0)),
              pl.BlockSpec((D, D), lambda i: (0, 0)),
              pl.BlockSpec((1, D), lambda i: (0, 0))],
    out_specs=pl.BlockSpec((1, D), lambda i: (0, 0)),
    out_shape=jax.ShapeDtypeStruct((1, D), jnp.float32),
    scratch_shapes=[pltpu.VMEM((1, D), jnp.float32),
                    pltpu.VMEM((1, D), jnp.float32)],
)


def kernel(n, edge_index, e, W_self1, W_neigh1, b1, W_self2, W_neigh2, b2,
           W_aw, b_aw, W_t1, b_t1, W_t2, b_t2):
    del e
    src = edge_index[0]
    dst = edge_index[1]
    pad = EPAD - E
    pad_i = jnp.arange(pad, dtype=jnp.int32)
    src_p = jnp.concatenate([src, pad_i % N])
    dst_p = jnp.concatenate([dst, N + pad_i % (NPAD - N)])
    zrows = jnp.zeros((K, D), jnp.float32)
    ones_mat = jnp.ones((N, D), jnp.float32)
    Wcat1 = jnp.concatenate([W_neigh1, W_self1], axis=1)
    Wcat2 = jnp.concatenate([W_neigh2, W_self2], axis=1)

    P1, S1 = _pre(n, Wcat1)
    (deg,) = _agg(ones_mat, src_p, dst_p, zrows)
    tok = (deg[0, 0] * 0.0).astype(jnp.int32)
    (agg1,) = _agg(P1, src_p + tok, dst_p, zrows)
    P2, S2 = _mid(S1, agg1[:NPAD], agg1[NPAD:], deg[:NPAD], deg[NPAD:],
                  b1.reshape(1, D), Wcat2)
    (agg2,) = _agg(P2, src_p, dst_p, zrows)
    y = _post(S2, agg2[:NPAD], agg2[NPAD:], deg[:NPAD], deg[NPAD:],
              b2.reshape(1, D),
              jnp.pad(W_aw, ((0, 0), (0, D - 1))),
              jnp.broadcast_to(b_aw.reshape(1, 1), (1, D)),
              W_t1, b_t1.reshape(1, D),
              jnp.pad(W_t2, ((0, 0), (0, D - 1))),
              jnp.broadcast_to(b_t2.reshape(1, 1), (1, D)))
    return y[:, 0:1]


# speedup vs baseline: 7.1229x; 1.5499x over previous
import jax
import jax.numpy as jnp
from jax import lax
from jax.experimental import pallas as pl
from jax.experimental.pallas import tpu as pltpu
from jax.experimental.pallas import tpu_sc as plsc

N = 10000
D = 128
NC = 2
NS = 16
NPAD = 10240
ROWS_PER_TILE = NPAD // NS
E = 320000
K = 128
EPT = 10240
EPAD = NC * NS * EPT
CHUNKS = EPT // K
RB = 1000
GRID = N // RB

_MESH = plsc.VectorSubcoreMesh(core_axis_name="c", subcore_axis_name="s")


def _agg_body(p_hbm, src_hbm, dst_hbm, zrows_hbm, agg_out,
              si0, di0, rw0, si1, di1, rw1, acc_sh, g0, g1):
    c = lax.axis_index("c")
    s = lax.axis_index("s")
    r0 = s * ROWS_PER_TILE
    pltpu.sync_copy(zrows_hbm, rw0)
    for j in range(ROWS_PER_TILE // K):
        pltpu.sync_copy(rw0, acc_sh.at[pl.ds(r0 + j * K, K)])
    plsc.subcore_barrier()
    ebase = (c * NS + s) * EPT

    bufs = ((si0, di0, rw0, g0), (si1, di1, rw1, g1))

    pltpu.sync_copy(src_hbm.at[pl.ds(ebase, K)], si0)
    pltpu.sync_copy(dst_hbm.at[pl.ds(ebase, K)], di0)
    pltpu.async_copy(p_hbm.at[si0], rw0, g0)

    def pair(j, carry):
        for b in (0, 1):
            i = 2 * j + b
            si, di, rw, g = bufs[b]
            nsi, ndi, nrw, ng = bufs[1 - b]

            def prep():
                nbase = ebase + (i + 1) * K
                pltpu.sync_copy(src_hbm.at[pl.ds(nbase, K)], nsi)
                pltpu.sync_copy(dst_hbm.at[pl.ds(nbase, K)], ndi)
                pltpu.async_copy(p_hbm.at[nsi], nrw, ng)

            if b == 0:
                prep()
            else:
                pl.when(j < CHUNKS // 2 - 1)(prep)
            pltpu.make_async_copy(p_hbm.at[si], rw, g).wait()
            pltpu.sync_copy(rw, acc_sh.at[di], add=True)
        return carry

    lax.fori_loop(0, CHUNKS // 2, pair, 0)
    plsc.subcore_barrier()
    obase = c * NPAD + r0
    for j in range(ROWS_PER_TILE // K):
        pltpu.sync_copy(acc_sh.at[pl.ds(r0 + j * K, K)], rw0)
        pltpu.sync_copy(rw0, agg_out.at[pl.ds(obase + j * K, K)])


_agg = pl.kernel(
    _agg_body,
    mesh=_MESH,
    out_type=[jax.ShapeDtypeStruct((NC * NPAD, D), jnp.float32)],
    scratch_types=[
        pltpu.VMEM((K,), jnp.int32),
        pltpu.VMEM((K,), jnp.int32),
        pltpu.VMEM((K, D), jnp.float32),
        pltpu.VMEM((K,), jnp.int32),
        pltpu.VMEM((K,), jnp.int32),
        pltpu.VMEM((K, D), jnp.float32),
        pltpu.VMEM_SHARED((NPAD, D), jnp.float32),
        pltpu.SemaphoreType.DMA,
        pltpu.SemaphoreType.DMA,
    ],
)


def _pre_body(x_ref, w_ref, p_ref, s_ref):
    h = jnp.dot(x_ref[...], w_ref[...], preferred_element_type=jnp.float32)
    p_ref[...] = h[:, :D]
    s_ref[...] = h[:, D:]


_pre = pl.pallas_call(
    _pre_body,
    grid=(GRID,),
    in_specs=[pl.BlockSpec((RB, D), lambda i: (i, 0)),
              pl.BlockSpec((D, 2 * D), lambda i: (0, 0))],
    out_specs=[pl.BlockSpec((RB, D), lambda i: (i, 0))] * 2,
    out_shape=[jax.ShapeDtypeStruct((N, D), jnp.float32)] * 2,
)


def _mid_body(s1_ref, a0_ref, a1_ref, d0_ref, d1_ref, b1_ref, w_ref,
              p_ref, s_ref):
    deg = d0_ref[:, 0:1] + d1_ref[:, 0:1]
    hn = (a0_ref[...] + a1_ref[...]) / jnp.maximum(deg, 1.0)
    h = jax.nn.sigmoid(s1_ref[...] + hn + b1_ref[...])
    hw = jnp.dot(h, w_ref[...], preferred_element_type=jnp.float32)
    p_ref[...] = hw[:, :D]
    s_ref[...] = hw[:, D:]


_mid = pl.pallas_call(
    _mid_body,
    grid=(GRID,),
    in_specs=[pl.BlockSpec((RB, D), lambda i: (i, 0)),
              pl.BlockSpec((RB, D), lambda i: (i, 0)),
              pl.BlockSpec((RB, D), lambda i: (i, 0)),
              pl.BlockSpec((RB, D), lambda i: (i, 0)),
              pl.BlockSpec((RB, D), lambda i: (i, 0)),
              pl.BlockSpec((1, D), lambda i: (0, 0)),
              pl.BlockSpec((D, 2 * D), lambda i: (0, 0))],
    out_specs=[pl.BlockSpec((RB, D), lambda i: (i, 0))] * 2,
    out_shape=[jax.ShapeDtypeStruct((N, D), jnp.float32)] * 2,
)


def _post_body(s2_ref, a0_ref, a1_ref, d0_ref, d1_ref, b2_ref, waw_ref,
               baw_ref, wt1_ref, bt1_ref, wt2_ref, bt2_ref, out_ref,
               hsum_ref, hmax_ref):
    i = pl.program_id(0)
    deg = d0_ref[:, 0:1] + d1_ref[:, 0:1]
    hn = (a0_ref[...] + a1_ref[...]) / jnp.maximum(deg, 1.0)
    h = jax.nn.sigmoid(s2_ref[...] + hn + b2_ref[...])
    w = jax.nn.sigmoid(
        jnp.dot(h, waw_ref[...], preferred_element_type=jnp.float32)[:, 0:1]
        + baw_ref[0, 0])
    psum = jnp.sum(w * h, axis=0, keepdims=True)
    pmax = jnp.max(h, axis=0, keepdims=True)

    @pl.when(i == 0)
    def _():
        hsum_ref[...] = psum
        hmax_ref[...] = pmax

    @pl.when(i > 0)
    def _():
        hsum_ref[...] = hsum_ref[...] + psum
        hmax_ref[...] = jnp.maximum(hmax_ref[...], pmax)

    @pl.when(i == GRID - 1)
    def _():
        g = jnp.concatenate([hsum_ref[...], hmax_ref[...]], axis=1)
        y1 = jax.nn.sigmoid(
            jnp.dot(g, wt1_ref[...], preferred_element_type=jnp.float32)
            + bt1_ref[...])
        y2 = jax.nn.sigmoid(
            jnp.dot(y1, wt2_ref[...], preferred_element_type=jnp.float32)[:, 0:1]
            + bt2_ref[0, 0])
        out_ref[...] = jnp.broadcast_to(y2, (1, D))


_post = pl.pallas_call(
    _post_body,
    grid=(GRID,),
    in_specs=[pl.BlockSpec((RB, D), lambda i: (i, 0)),
              pl.BlockSpec((RB, D), lambda i: (i, 0)),
              pl.BlockSpec((RB, D), lambda i: (i, 0)),
              pl.BlockSpec((RB, D), lambda i: (i, 0)),
              pl.BlockSpec((RB, D), lambda i: (i, 0)),
              pl.BlockSpec((1, D), lambda i: (0, 0)),
              pl.BlockSpec((D, D), lambda i: (0, 0)),
              pl.BlockSpec((1, D), lambda i: (0, 0)),
              pl.BlockSpec((2 * D, D), lambda i: (0, 0)),
              pl.BlockSpec((1, D), lambda i: (0, 0)),
              pl.BlockSpec((D, D), lambda i: (0, 0)),
              pl.BlockSpec((1, D), lambda i: (0, 0))],
    out_specs=pl.BlockSpec((1, D), lambda i: (0, 0)),
    out_shape=jax.ShapeDtypeStruct((1, D), jnp.float32),
    scratch_shapes=[pltpu.VMEM((1, D), jnp.float32),
                    pltpu.VMEM((1, D), jnp.float32)],
)


def kernel(n, edge_index, e, W_self1, W_neigh1, b1, W_self2, W_neigh2, b2,
           W_aw, b_aw, W_t1, b_t1, W_t2, b_t2):
    del e
    src = edge_index[0]
    dst = edge_index[1]
    pad = EPAD - E
    pad_i = jnp.arange(pad, dtype=jnp.int32)
    src_p = jnp.concatenate([src, pad_i % N])
    dst_p = jnp.concatenate([dst, N + pad_i % (NPAD - N)])
    zrows = jnp.zeros((K, D), jnp.float32)
    ones_mat = jnp.ones((N, D), jnp.float32)
    Wcat1 = jnp.concatenate([W_neigh1, W_self1], axis=1)
    Wcat2 = jnp.concatenate([W_neigh2, W_self2], axis=1)

    P1, S1 = _pre(n, Wcat1)
    (deg,) = _agg(ones_mat, src_p, dst_p, zrows)
    tok = (deg[0, 0] * 0.0).astype(jnp.int32)
    (agg1,) = _agg(P1, src_p + tok, dst_p, zrows)
    P2, S2 = _mid(S1, agg1[:NPAD], agg1[NPAD:], deg[:NPAD], deg[NPAD:],
                  b1.reshape(1, D), Wcat2)
    (agg2,) = _agg(P2, src_p, dst_p, zrows)
    y = _post(S2, agg2[:NPAD], agg2[NPAD:], deg[:NPAD], deg[NPAD:],
              b2.reshape(1, D),
              jnp.pad(W_aw, ((0, 0), (0, D - 1))),
              jnp.broadcast_to(b_aw.reshape(1, 1), (1, D)),
              W_t1, b_t1.reshape(1, D),
              jnp.pad(W_t2, ((0, 0), (0, D - 1))),
              jnp.broadcast_to(b_t2.reshape(1, 1), (1, D)))
    return y[:, 0:1]


# speedup vs baseline: 8.2331x; 1.1559x over previous
import jax
import jax.numpy as jnp
from jax import lax
from jax.experimental import pallas as pl
from jax.experimental.pallas import tpu as pltpu
from jax.experimental.pallas import tpu_sc as plsc

N = 10000
D = 128
NC = 2
NS = 16
NPAD = 10240
ROWS_PER_TILE = NPAD // NS
E = 320000
K = 128
EPT = 10240
EPAD = NC * NS * EPT
CHUNKS = EPT // K
RB = 1000
GRID = N // RB

_MESH = plsc.VectorSubcoreMesh(core_axis_name="c", subcore_axis_name="s")


def _agg_body(p_hbm, ei_hbm, zrows_hbm, agg_out,
              ix0, rw0, ix1, rw1, acc_sh, g0, g1):
    c = lax.axis_index("c")
    s = lax.axis_index("s")
    r0 = s * ROWS_PER_TILE
    pltpu.sync_copy(zrows_hbm, rw0)
    for j in range(ROWS_PER_TILE // K):
        pltpu.sync_copy(rw0, acc_sh.at[pl.ds(r0 + j * K, K)])
    plsc.subcore_barrier()
    cbase = (c * NS + s) * CHUNKS

    bufs = ((ix0, rw0, g0), (ix1, rw1, g1))

    pltpu.sync_copy(ei_hbm.at[cbase], ix0)
    pltpu.async_copy(p_hbm.at[ix0.at[0]], rw0, g0)

    def pair(j, carry):
        for b in (0, 1):
            i = 2 * j + b
            ix, rw, g = bufs[b]
            nix, nrw, ng = bufs[1 - b]

            def prep():
                pltpu.sync_copy(ei_hbm.at[cbase + i + 1], nix)
                pltpu.async_copy(p_hbm.at[nix.at[0]], nrw, ng)

            if b == 0:
                prep()
            else:
                pl.when(j < CHUNKS // 2 - 1)(prep)
            pltpu.make_async_copy(p_hbm.at[ix.at[0]], rw, g).wait()
            pltpu.sync_copy(rw, acc_sh.at[ix.at[1]], add=True)
        return carry

    lax.fori_loop(0, CHUNKS // 2, pair, 0)
    plsc.subcore_barrier()
    obase = c * NPAD + r0
    for j in range(ROWS_PER_TILE // K):
        pltpu.sync_copy(acc_sh.at[pl.ds(r0 + j * K, K)], rw0)
        pltpu.sync_copy(rw0, agg_out.at[pl.ds(obase + j * K, K)])


_agg = pl.kernel(
    _agg_body,
    mesh=_MESH,
    out_type=[jax.ShapeDtypeStruct((NC * NPAD, D), jnp.float32)],
    scratch_types=[
        pltpu.VMEM((2, K), jnp.int32),
        pltpu.VMEM((K, D), jnp.float32),
        pltpu.VMEM((2, K), jnp.int32),
        pltpu.VMEM((K, D), jnp.float32),
        pltpu.VMEM_SHARED((NPAD, D), jnp.float32),
        pltpu.SemaphoreType.DMA,
        pltpu.SemaphoreType.DMA,
    ],
)


def _pre_body(x_ref, w_ref, p_ref, s_ref):
    h = jnp.dot(x_ref[...], w_ref[...], preferred_element_type=jnp.float32)
    p_ref[...] = h[:, :D]
    s_ref[...] = h[:, D:]


_pre = pl.pallas_call(
    _pre_body,
    grid=(GRID,),
    in_specs=[pl.BlockSpec((RB, D), lambda i: (i, 0)),
              pl.BlockSpec((D, 2 * D), lambda i: (0, 0))],
    out_specs=[pl.BlockSpec((RB, D), lambda i: (i, 0))] * 2,
    out_shape=[jax.ShapeDtypeStruct((N, D), jnp.float32)] * 2,
)


def _mid_body(s1_ref, a0_ref, a1_ref, d0_ref, d1_ref, b1_ref, w_ref,
              p_ref, s_ref):
    deg = d0_ref[:, 0:1] + d1_ref[:, 0:1]
    hn = (a0_ref[...] + a1_ref[...]) / jnp.maximum(deg, 1.0)
    h = jax.nn.sigmoid(s1_ref[...] + hn + b1_ref[...])
    hw = jnp.dot(h, w_ref[...], preferred_element_type=jnp.float32)
    p_ref[...] = hw[:, :D]
    s_ref[...] = hw[:, D:]


_mid = pl.pallas_call(
    _mid_body,
    grid=(GRID,),
    in_specs=[pl.BlockSpec((RB, D), lambda i: (i, 0)),
              pl.BlockSpec((RB, D), lambda i: (i, 0)),
              pl.BlockSpec((RB, D), lambda i: (i, 0)),
              pl.BlockSpec((RB, D), lambda i: (i, 0)),
              pl.BlockSpec((RB, D), lambda i: (i, 0)),
              pl.BlockSpec((1, D), lambda i: (0, 0)),
              pl.BlockSpec((D, 2 * D), lambda i: (0, 0))],
    out_specs=[pl.BlockSpec((RB, D), lambda i: (i, 0))] * 2,
    out_shape=[jax.ShapeDtypeStruct((N, D), jnp.float32)] * 2,
)


def _post_body(s2_ref, a0_ref, a1_ref, d0_ref, d1_ref, b2_ref, waw_ref,
               baw_ref, wt1_ref, bt1_ref, wt2_ref, bt2_ref, out_ref,
               hsum_ref, hmax_ref):
    i = pl.program_id(0)
    deg = d0_ref[:, 0:1] + d1_ref[:, 0:1]
    hn = (a0_ref[...] + a1_ref[...]) / jnp.maximum(deg, 1.0)
    h = jax.nn.sigmoid(s2_ref[...] + hn + b2_ref[...])
    w = jax.nn.sigmoid(
        jnp.dot(h, waw_ref[...], preferred_element_type=jnp.float32)[:, 0:1]
        + baw_ref[0, 0])
    psum = jnp.sum(w * h, axis=0, keepdims=True)
    pmax = jnp.max(h, axis=0, keepdims=True)

    @pl.when(i == 0)
    def _():
        hsum_ref[...] = psum
        hmax_ref[...] = pmax

    @pl.when(i > 0)
    def _():
        hsum_ref[...] = hsum_ref[...] + psum
        hmax_ref[...] = jnp.maximum(hmax_ref[...], pmax)

    @pl.when(i == GRID - 1)
    def _():
        g = jnp.concatenate([hsum_ref[...], hmax_ref[...]], axis=1)
        y1 = jax.nn.sigmoid(
            jnp.dot(g, wt1_ref[...], preferred_element_type=jnp.float32)
            + bt1_ref[...])
        y2 = jax.nn.sigmoid(
            jnp.dot(y1, wt2_ref[...], preferred_element_type=jnp.float32)[:, 0:1]
            + bt2_ref[0, 0])
        out_ref[...] = jnp.broadcast_to(y2, (1, D))


_post = pl.pallas_call(
    _post_body,
    grid=(GRID,),
    in_specs=[pl.BlockSpec((RB, D), lambda i: (i, 0)),
              pl.BlockSpec((RB, D), lambda i: (i, 0)),
              pl.BlockSpec((RB, D), lambda i: (i, 0)),
              pl.BlockSpec((RB, D), lambda i: (i, 0)),
              pl.BlockSpec((RB, D), lambda i: (i, 0)),
              pl.BlockSpec((1, D), lambda i: (0, 0)),
              pl.BlockSpec((D, D), lambda i: (0, 0)),
              pl.BlockSpec((1, D), lambda i: (0, 0)),
              pl.BlockSpec((2 * D, D), lambda i: (0, 0)),
              pl.BlockSpec((1, D), lambda i: (0, 0)),
              pl.BlockSpec((D, D), lambda i: (0, 0)),
              pl.BlockSpec((1, D), lambda i: (0, 0))],
    out_specs=pl.BlockSpec((1, D), lambda i: (0, 0)),
    out_shape=jax.ShapeDtypeStruct((1, D), jnp.float32),
    scratch_shapes=[pltpu.VMEM((1, D), jnp.float32),
                    pltpu.VMEM((1, D), jnp.float32)],
)


def kernel(n, edge_index, e, W_self1, W_neigh1, b1, W_self2, W_neigh2, b2,
           W_aw, b_aw, W_t1, b_t1, W_t2, b_t2):
    del e
    src = edge_index[0]
    dst = edge_index[1]
    pad = EPAD - E
    pad_i = jnp.arange(pad, dtype=jnp.int32)
    src_p = jnp.concatenate([src, pad_i % N])
    dst_p = jnp.concatenate([dst, N + pad_i % (NPAD - N)])
    ei = jnp.stack([src_p.reshape(EPAD // K, K),
                    dst_p.reshape(EPAD // K, K)], axis=1)
    zrows = jnp.zeros((K, D), jnp.float32)
    ones_mat = jnp.ones((N, D), jnp.float32)
    Wcat1 = jnp.concatenate([W_neigh1, W_self1], axis=1)
    Wcat2 = jnp.concatenate([W_neigh2, W_self2], axis=1)

    P1, S1 = _pre(n, Wcat1)
    (deg,) = _agg(ones_mat, ei, zrows)
    tok = (deg[0, 0] * 0.0).astype(jnp.int32)
    (agg1,) = _agg(P1, ei + tok, zrows)
    P2, S2 = _mid(S1, agg1[:NPAD], agg1[NPAD:], deg[:NPAD], deg[NPAD:],
                  b1.reshape(1, D), Wcat2)
    (agg2,) = _agg(P2, ei, zrows)
    y = _post(S2, agg2[:NPAD], agg2[NPAD:], deg[:NPAD], deg[NPAD:],
              b2.reshape(1, D),
              jnp.pad(W_aw, ((0, 0), (0, D - 1))),
              jnp.broadcast_to(b_aw.reshape(1, 1), (1, D)),
              W_t1, b_t1.reshape(1, D),
              jnp.pad(W_t2, ((0, 0), (0, D - 1))),
              jnp.broadcast_to(b_t2.reshape(1, 1), (1, D)))
    return y[:, 0:1]


# speedup vs baseline: 9.2905x; 1.1284x over previous
import jax
import jax.numpy as jnp
from jax import lax
from jax.experimental import pallas as pl
from jax.experimental.pallas import tpu as pltpu
from jax.experimental.pallas import tpu_sc as plsc

N = 10000
D = 128
NC = 2
NS = 16
NPAD = 10240
ROWS_PER_TILE = NPAD // NS
E = 320000
K = 128
EPT = 10240
EPAD = NC * NS * EPT
CHUNKS = EPT // K
RB = 1000
GRID = N // RB

_MESH = plsc.VectorSubcoreMesh(core_axis_name="c", subcore_axis_name="s")


def _agg_body(p_hbm, ei_hbm, zrows_hbm, agg_out,
              ix0, rw0, ix1, rw1, acc_sh, g0, g1):
    c = lax.axis_index("c")
    s = lax.axis_index("s")
    r0 = s * ROWS_PER_TILE
    pltpu.sync_copy(zrows_hbm, rw0)
    for j in range(ROWS_PER_TILE // K):
        pltpu.sync_copy(rw0, acc_sh.at[pl.ds(r0 + j * K, K)])
    plsc.subcore_barrier()
    cbase = (c * NS + s) * CHUNKS

    bufs = ((ix0, rw0, g0), (ix1, rw1, g1))

    pltpu.sync_copy(ei_hbm.at[cbase], ix0)
    pltpu.async_copy(p_hbm.at[ix0.at[0]], rw0, g0)

    def pair(j, carry):
        for b in (0, 1):
            i = 2 * j + b
            ix, rw, g = bufs[b]
            nix, nrw, ng = bufs[1 - b]

            def prep():
                pltpu.sync_copy(ei_hbm.at[cbase + i + 1], nix)
                pltpu.async_copy(p_hbm.at[nix.at[0]], nrw, ng)

            if b == 0:
                prep()
            else:
                pl.when(j < CHUNKS // 2 - 1)(prep)
            pltpu.make_async_copy(p_hbm.at[ix.at[0]], rw, g).wait()
            pltpu.sync_copy(rw, acc_sh.at[ix.at[1]], add=True)
        return carry

    lax.fori_loop(0, CHUNKS // 2, pair, 0)
    plsc.subcore_barrier()
    obase = c * NPAD + r0
    for j in range(ROWS_PER_TILE // K):
        pltpu.sync_copy(acc_sh.at[pl.ds(r0 + j * K, K)], rw0)
        pltpu.sync_copy(rw0, agg_out.at[pl.ds(obase + j * K, K)])


_agg = pl.kernel(
    _agg_body,
    mesh=_MESH,
    out_type=[jax.ShapeDtypeStruct((NC * NPAD, D), jnp.float32)],
    scratch_types=[
        pltpu.VMEM((2, K), jnp.int32),
        pltpu.VMEM((K, D), jnp.float32),
        pltpu.VMEM((2, K), jnp.int32),
        pltpu.VMEM((K, D), jnp.float32),
        pltpu.VMEM_SHARED((NPAD, D), jnp.float32),
        pltpu.SemaphoreType.DMA,
        pltpu.SemaphoreType.DMA,
    ],
)


def _deg_body(ei_hbm, zrows_hbm, ones_hbm, deg_out,
              ix0, ix1, ones_v, acc_sh, i0, i1):
    c = lax.axis_index("c")
    s = lax.axis_index("s")
    r0 = s * ROWS_PER_TILE
    pltpu.sync_copy(zrows_hbm, ones_v)
    for j in range(ROWS_PER_TILE // K):
        pltpu.sync_copy(ones_v, acc_sh.at[pl.ds(r0 + j * K, K)])
    pltpu.sync_copy(ones_hbm, ones_v)
    plsc.subcore_barrier()
    cbase = (c * NS + s) * CHUNKS

    bufs = ((ix0, i0), (ix1, i1))
    pltpu.async_copy(ei_hbm.at[cbase], ix0, i0)

    def pair(j, carry):
        for b in (0, 1):
            i = 2 * j + b
            ix, isem = bufs[b]
            nix, nisem = bufs[1 - b]

            def prep():
                pltpu.async_copy(ei_hbm.at[cbase + i + 1], nix, nisem)

            if b == 0:
                prep()
            else:
                pl.when(j < CHUNKS // 2 - 1)(prep)
            pltpu.make_async_copy(ei_hbm.at[cbase], ix, isem).wait()
            pltpu.sync_copy(ones_v, acc_sh.at[ix.at[1]], add=True)
        return carry

    lax.fori_loop(0, CHUNKS // 2, pair, 0)
    plsc.subcore_barrier()
    obase = c * NPAD + r0
    for j in range(ROWS_PER_TILE // K):
        pltpu.sync_copy(acc_sh.at[pl.ds(r0 + j * K, K)], ones_v)
        pltpu.sync_copy(ones_v, deg_out.at[pl.ds(obase + j * K, K)])


_deg = pl.kernel(
    _deg_body,
    mesh=_MESH,
    out_type=[jax.ShapeDtypeStruct((NC * NPAD, D), jnp.float32)],
    scratch_types=[
        pltpu.VMEM((2, K), jnp.int32),
        pltpu.VMEM((2, K), jnp.int32),
        pltpu.VMEM((K, D), jnp.float32),
        pltpu.VMEM_SHARED((NPAD, D), jnp.float32),
        pltpu.SemaphoreType.DMA,
        pltpu.SemaphoreType.DMA,
    ],
)


def _pre_body(x_ref, w_ref, p_ref, s_ref):
    h = jnp.dot(x_ref[...], w_ref[...], preferred_element_type=jnp.float32)
    p_ref[...] = h[:, :D]
    s_ref[...] = h[:, D:]


_pre = pl.pallas_call(
    _pre_body,
    grid=(GRID,),
    in_specs=[pl.BlockSpec((RB, D), lambda i: (i, 0)),
              pl.BlockSpec((D, 2 * D), lambda i: (0, 0))],
    out_specs=[pl.BlockSpec((RB, D), lambda i: (i, 0))] * 2,
    out_shape=[jax.ShapeDtypeStruct((N, D), jnp.float32)] * 2,
)


def _mid_body(s1_ref, a0_ref, a1_ref, d0_ref, d1_ref, b1_ref, w_ref,
              p_ref, s_ref):
    deg = d0_ref[:, 0:1] + d1_ref[:, 0:1]
    hn = (a0_ref[...] + a1_ref[...]) / jnp.maximum(deg, 1.0)
    h = jax.nn.sigmoid(s1_ref[...] + hn + b1_ref[...])
    hw = jnp.dot(h, w_ref[...], preferred_element_type=jnp.float32)
    p_ref[...] = hw[:, :D]
    s_ref[...] = hw[:, D:]


_mid = pl.pallas_call(
    _mid_body,
    grid=(GRID,),
    in_specs=[pl.BlockSpec((RB, D), lambda i: (i, 0)),
              pl.BlockSpec((RB, D), lambda i: (i, 0)),
              pl.BlockSpec((RB, D), lambda i: (i, 0)),
              pl.BlockSpec((RB, D), lambda i: (i, 0)),
              pl.BlockSpec((RB, D), lambda i: (i, 0)),
              pl.BlockSpec((1, D), lambda i: (0, 0)),
              pl.BlockSpec((D, 2 * D), lambda i: (0, 0))],
    out_specs=[pl.BlockSpec((RB, D), lambda i: (i, 0))] * 2,
    out_shape=[jax.ShapeDtypeStruct((N, D), jnp.float32)] * 2,
)


def _post_body(s2_ref, a0_ref, a1_ref, d0_ref, d1_ref, b2_ref, waw_ref,
               baw_ref, wt1_ref, bt1_ref, wt2_ref, bt2_ref, out_ref,
               hsum_ref, hmax_ref):
    i = pl.program_id(0)
    deg = d0_ref[:, 0:1] + d1_ref[:, 0:1]
    hn = (a0_ref[...] + a1_ref[...]) / jnp.maximum(deg, 1.0)
    h = jax.nn.sigmoid(s2_ref[...] + hn + b2_ref[...])
    w = jax.nn.sigmoid(
        jnp.dot(h, waw_ref[...], preferred_element_type=jnp.float32)[:, 0:1]
        + baw_ref[0, 0])
    psum = jnp.sum(w * h, axis=0, keepdims=True)
    pmax = jnp.max(h, axis=0, keepdims=True)

    @pl.when(i == 0)
    def _():
        hsum_ref[...] = psum
        hmax_ref[...] = pmax

    @pl.when(i > 0)
    def _():
        hsum_ref[...] = hsum_ref[...] + psum
        hmax_ref[...] = jnp.maximum(hmax_ref[...], pmax)

    @pl.when(i == GRID - 1)
    def _():
        g = jnp.concatenate([hsum_ref[...], hmax_ref[...]], axis=1)
        y1 = jax.nn.sigmoid(
            jnp.dot(g, wt1_ref[...], preferred_element_type=jnp.float32)
            + bt1_ref[...])
        y2 = jax.nn.sigmoid(
            jnp.dot(y1, wt2_ref[...], preferred_element_type=jnp.float32)[:, 0:1]
            + bt2_ref[0, 0])
        out_ref[...] = jnp.broadcast_to(y2, (1, D))


_post = pl.pallas_call(
    _post_body,
    grid=(GRID,),
    in_specs=[pl.BlockSpec((RB, D), lambda i: (i, 0)),
              pl.BlockSpec((RB, D), lambda i: (i, 0)),
              pl.BlockSpec((RB, D), lambda i: (i, 0)),
              pl.BlockSpec((RB, D), lambda i: (i, 0)),
              pl.BlockSpec((RB, D), lambda i: (i, 0)),
              pl.BlockSpec((1, D), lambda i: (0, 0)),
              pl.BlockSpec((D, D), lambda i: (0, 0)),
              pl.BlockSpec((1, D), lambda i: (0, 0)),
              pl.BlockSpec((2 * D, D), lambda i: (0, 0)),
              pl.BlockSpec((1, D), lambda i: (0, 0)),
              pl.BlockSpec((D, D), lambda i: (0, 0)),
              pl.BlockSpec((1, D), lambda i: (0, 0))],
    out_specs=pl.BlockSpec((1, D), lambda i: (0, 0)),
    out_shape=jax.ShapeDtypeStruct((1, D), jnp.float32),
    scratch_shapes=[pltpu.VMEM((1, D), jnp.float32),
                    pltpu.VMEM((1, D), jnp.float32)],
)


def kernel(n, edge_index, e, W_self1, W_neigh1, b1, W_self2, W_neigh2, b2,
           W_aw, b_aw, W_t1, b_t1, W_t2, b_t2):
    del e
    src = edge_index[0]
    dst = edge_index[1]
    pad = EPAD - E
    pad_i = jnp.arange(pad, dtype=jnp.int32)
    src_p = jnp.concatenate([src, pad_i % N])
    dst_p = jnp.concatenate([dst, N + pad_i % (NPAD - N)])
    ei = jnp.stack([src_p.reshape(EPAD // K, K),
                    dst_p.reshape(EPAD // K, K)], axis=1)
    zrows = jnp.zeros((K, D), jnp.float32)
    ones_rows = jnp.ones((K, D), jnp.float32)
    Wcat1 = jnp.concatenate([W_neigh1, W_self1], axis=1)
    Wcat2 = jnp.concatenate([W_neigh2, W_self2], axis=1)

    P1, S1 = _pre(n, Wcat1)
    (deg,) = _deg(ei, zrows, ones_rows)
    tok = (deg[0, 0] * 0.0).astype(jnp.int32)
    (agg1,) = _agg(P1, ei + tok, zrows)
    P2, S2 = _mid(S1, agg1[:NPAD], agg1[NPAD:], deg[:NPAD], deg[NPAD:],
                  b1.reshape(1, D), Wcat2)
    (agg2,) = _agg(P2, ei, zrows)
    y = _post(S2, agg2[:NPAD], agg2[NPAD:], deg[:NPAD], deg[NPAD:],
              b2.reshape(1, D),
              jnp.pad(W_aw, ((0, 0), (0, D - 1))),
              jnp.broadcast_to(b_aw.reshape(1, 1), (1, D)),
              W_t1, b_t1.reshape(1, D),
              jnp.pad(W_t2, ((0, 0), (0, D - 1))),
              jnp.broadcast_to(b_t2.reshape(1, 1), (1, D)))
    return y[:, 0:1]


# speedup vs baseline: 10.1120x; 1.0884x over previous
import jax
import jax.numpy as jnp
from jax import lax
from jax.experimental import pallas as pl
from jax.experimental.pallas import tpu as pltpu
from jax.experimental.pallas import tpu_sc as plsc

N = 10000
D = 128
NC = 2
NS = 16
NPAD = 10240
ROWS_PER_TILE = NPAD // NS
E = 320000
K = 128
EPT = 10240
EPAD = NC * NS * EPT
CHUNKS = EPT // K
RB = 1000
GRID = N // RB

_MESH = plsc.VectorSubcoreMesh(core_axis_name="c", subcore_axis_name="s")


def _agg_body(p_hbm, ei_hbm, zrows_hbm, agg_out,
              ix0, ix1, ix2, ix3, rw0, rw1, acc_sh,
              i0, i1, i2, i3, g0, g1):
    c = lax.axis_index("c")
    s = lax.axis_index("s")
    r0 = s * ROWS_PER_TILE
    pltpu.sync_copy(zrows_hbm, rw0)
    for j in range(ROWS_PER_TILE // K):
        pltpu.sync_copy(rw0, acc_sh.at[pl.ds(r0 + j * K, K)])
    plsc.subcore_barrier()
    cbase = (c * NS + s) * CHUNKS

    ixs = (ix0, ix1, ix2, ix3)
    isems = (i0, i1, i2, i3)
    rws = (rw0, rw1)
    gsems = (g0, g1)

    pltpu.async_copy(ei_hbm.at[cbase], ix0, i0)
    pltpu.async_copy(ei_hbm.at[cbase + 1], ix1, i1)
    pltpu.make_async_copy(ei_hbm.at[cbase], ix0, i0).wait()
    pltpu.async_copy(p_hbm.at[ix0.at[0]], rw0, g0)

    def quad(j, carry):
        for b in (0, 1, 2, 3):
            i = 4 * j + b
            ix = ixs[b % 4]
            rw, g = rws[b % 2], gsems[b % 2]
            nix, nisem = ixs[(b + 2) % 4], isems[(b + 2) % 4]
            pix, pisem = ixs[(b + 1) % 4], isems[(b + 1) % 4]
            nrw, ng = rws[(b + 1) % 2], gsems[(b + 1) % 2]

            def prep_idx():
                pltpu.async_copy(ei_hbm.at[cbase + i + 2], nix, nisem)

            def prep_gather():
                pltpu.make_async_copy(ei_hbm.at[cbase], pix, pisem).wait()
                pltpu.async_copy(p_hbm.at[pix.at[0]], nrw, ng)

            if b < 2:
                prep_idx()
            else:
                pl.when(j < CHUNKS // 4 - 1)(prep_idx)
            if b < 3:
                prep_gather()
            else:
                pl.when(j < CHUNKS // 4 - 1)(prep_gather)
            pltpu.make_async_copy(p_hbm.at[ix.at[0]], rw, g).wait()
            pltpu.sync_copy(rw, acc_sh.at[ix.at[1]], add=True)
        return carry

    lax.fori_loop(0, CHUNKS // 4, quad, 0)
    plsc.subcore_barrier()
    obase = c * NPAD + r0
    for j in range(ROWS_PER_TILE // K):
        pltpu.sync_copy(acc_sh.at[pl.ds(r0 + j * K, K)], rw0)
        pltpu.sync_copy(rw0, agg_out.at[pl.ds(obase + j * K, K)])


_agg = pl.kernel(
    _agg_body,
    mesh=_MESH,
    out_type=[jax.ShapeDtypeStruct((NC * NPAD, D), jnp.float32)],
    scratch_types=[
        pltpu.VMEM((2, K), jnp.int32),
        pltpu.VMEM((2, K), jnp.int32),
        pltpu.VMEM((2, K), jnp.int32),
        pltpu.VMEM((2, K), jnp.int32),
        pltpu.VMEM((K, D), jnp.float32),
        pltpu.VMEM((K, D), jnp.float32),
        pltpu.VMEM_SHARED((NPAD, D), jnp.float32),
        pltpu.SemaphoreType.DMA,
        pltpu.SemaphoreType.DMA,
        pltpu.SemaphoreType.DMA,
        pltpu.SemaphoreType.DMA,
        pltpu.SemaphoreType.DMA,
        pltpu.SemaphoreType.DMA,
    ],
)


def _deg_body(ei_hbm, zrows_hbm, ones_hbm, deg_out,
              ix0, ix1, ones_v, acc_sh, i0, i1):
    c = lax.axis_index("c")
    s = lax.axis_index("s")
    r0 = s * ROWS_PER_TILE
    pltpu.sync_copy(zrows_hbm, ones_v)
    for j in range(ROWS_PER_TILE // K):
        pltpu.sync_copy(ones_v, acc_sh.at[pl.ds(r0 + j * K, K)])
    pltpu.sync_copy(ones_hbm, ones_v)
    plsc.subcore_barrier()
    cbase = (c * NS + s) * CHUNKS

    bufs = ((ix0, i0), (ix1, i1))
    pltpu.async_copy(ei_hbm.at[cbase], ix0, i0)

    def pair(j, carry):
        for b in (0, 1):
            i = 2 * j + b
            ix, isem = bufs[b]
            nix, nisem = bufs[1 - b]

            def prep():
                pltpu.async_copy(ei_hbm.at[cbase + i + 1], nix, nisem)

            if b == 0:
                prep()
            else:
                pl.when(j < CHUNKS // 2 - 1)(prep)
            pltpu.make_async_copy(ei_hbm.at[cbase], ix, isem).wait()
            pltpu.sync_copy(ones_v, acc_sh.at[ix.at[1]], add=True)
        return carry

    lax.fori_loop(0, CHUNKS // 2, pair, 0)
    plsc.subcore_barrier()
    obase = c * NPAD + r0
    for j in range(ROWS_PER_TILE // K):
        pltpu.sync_copy(acc_sh.at[pl.ds(r0 + j * K, K)], ones_v)
        pltpu.sync_copy(ones_v, deg_out.at[pl.ds(obase + j * K, K)])


_deg = pl.kernel(
    _deg_body,
    mesh=_MESH,
    out_type=[jax.ShapeDtypeStruct((NC * NPAD, D), jnp.float32)],
    scratch_types=[
        pltpu.VMEM((2, K), jnp.int32),
        pltpu.VMEM((2, K), jnp.int32),
        pltpu.VMEM((K, D), jnp.float32),
        pltpu.VMEM_SHARED((NPAD, D), jnp.float32),
        pltpu.SemaphoreType.DMA,
        pltpu.SemaphoreType.DMA,
    ],
)


def _pre_body(x_ref, w_ref, p_ref, s_ref):
    h = jnp.dot(x_ref[...], w_ref[...], preferred_element_type=jnp.float32)
    p_ref[...] = h[:, :D]
    s_ref[...] = h[:, D:]


_pre = pl.pallas_call(
    _pre_body,
    grid=(GRID,),
    in_specs=[pl.BlockSpec((RB, D), lambda i: (i, 0)),
              pl.BlockSpec((D, 2 * D), lambda i: (0, 0))],
    out_specs=[pl.BlockSpec((RB, D), lambda i: (i, 0))] * 2,
    out_shape=[jax.ShapeDtypeStruct((N, D), jnp.float32)] * 2,
)


def _mid_body(s1_ref, a0_ref, a1_ref, d0_ref, d1_ref, b1_ref, w_ref,
              p_ref, s_ref):
    deg = d0_ref[:, 0:1] + d1_ref[:, 0:1]
    hn = (a0_ref[...] + a1_ref[...]) / jnp.maximum(deg, 1.0)
    h = jax.nn.sigmoid(s1_ref[...] + hn + b1_ref[...])
    hw = jnp.dot(h, w_ref[...], preferred_element_type=jnp.float32)
    p_ref[...] = hw[:, :D]
    s_ref[...] = hw[:, D:]


_mid = pl.pallas_call(
    _mid_body,
    grid=(GRID,),
    in_specs=[pl.BlockSpec((RB, D), lambda i: (i, 0)),
              pl.BlockSpec((RB, D), lambda i: (i, 0)),
              pl.BlockSpec((RB, D), lambda i: (i, 0)),
              pl.BlockSpec((RB, D), lambda i: (i, 0)),
              pl.BlockSpec((RB, D), lambda i: (i, 0)),
              pl.BlockSpec((1, D), lambda i: (0, 0)),
              pl.BlockSpec((D, 2 * D), lambda i: (0, 0))],
    out_specs=[pl.BlockSpec((RB, D), lambda i: (i, 0))] * 2,
    out_shape=[jax.ShapeDtypeStruct((N, D), jnp.float32)] * 2,
)


def _post_body(s2_ref, a0_ref, a1_ref, d0_ref, d1_ref, b2_ref, waw_ref,
               baw_ref, wt1_ref, bt1_ref, wt2_ref, bt2_ref, out_ref,
               hsum_ref, hmax_ref):
    i = pl.program_id(0)
    deg = d0_ref[:, 0:1] + d1_ref[:, 0:1]
    hn = (a0_ref[...] + a1_ref[...]) / jnp.maximum(deg, 1.0)
    h = jax.nn.sigmoid(s2_ref[...] + hn + b2_ref[...])
    w = jax.nn.sigmoid(
        jnp.dot(h, waw_ref[...], preferred_element_type=jnp.float32)[:, 0:1]
        + baw_ref[0, 0])
    psum = jnp.sum(w * h, axis=0, keepdims=True)
    pmax = jnp.max(h, axis=0, keepdims=True)

    @pl.when(i == 0)
    def _():
        hsum_ref[...] = psum
        hmax_ref[...] = pmax

    @pl.when(i > 0)
    def _():
        hsum_ref[...] = hsum_ref[...] + psum
        hmax_ref[...] = jnp.maximum(hmax_ref[...], pmax)

    @pl.when(i == GRID - 1)
    def _():
        g = jnp.concatenate([hsum_ref[...], hmax_ref[...]], axis=1)
        y1 = jax.nn.sigmoid(
            jnp.dot(g, wt1_ref[...], preferred_element_type=jnp.float32)
            + bt1_ref[...])
        y2 = jax.nn.sigmoid(
            jnp.dot(y1, wt2_ref[...], preferred_element_type=jnp.float32)[:, 0:1]
            + bt2_ref[0, 0])
        out_ref[...] = jnp.broadcast_to(y2, (1, D))


_post = pl.pallas_call(
    _post_body,
    grid=(GRID,),
    in_specs=[pl.BlockSpec((RB, D), lambda i: (i, 0)),
              pl.BlockSpec((RB, D), lambda i: (i, 0)),
              pl.BlockSpec((RB, D), lambda i: (i, 0)),
              pl.BlockSpec((RB, D), lambda i: (i, 0)),
              pl.BlockSpec((RB, D), lambda i: (i, 0)),
              pl.BlockSpec((1, D), lambda i: (0, 0)),
              pl.BlockSpec((D, D), lambda i: (0, 0)),
              pl.BlockSpec((1, D), lambda i: (0, 0)),
              pl.BlockSpec((2 * D, D), lambda i: (0, 0)),
              pl.BlockSpec((1, D), lambda i: (0, 0)),
              pl.BlockSpec((D, D), lambda i: (0, 0)),
              pl.BlockSpec((1, D), lambda i: (0, 0))],
    out_specs=pl.BlockSpec((1, D), lambda i: (0, 0)),
    out_shape=jax.ShapeDtypeStruct((1, D), jnp.float32),
    scratch_shapes=[pltpu.VMEM((1, D), jnp.float32),
                    pltpu.VMEM((1, D), jnp.float32)],
)


def kernel(n, edge_index, e, W_self1, W_neigh1, b1, W_self2, W_neigh2, b2,
           W_aw, b_aw, W_t1, b_t1, W_t2, b_t2):
    del e
    src = edge_index[0]
    dst = edge_index[1]
    pad = EPAD - E
    pad_i = jnp.arange(pad, dtype=jnp.int32)
    src_p = jnp.concatenate([src, pad_i % N])
    dst_p = jnp.concatenate([dst, N + pad_i % (NPAD - N)])
    ei = jnp.stack([src_p.reshape(EPAD // K, K),
                    dst_p.reshape(EPAD // K, K)], axis=1)
    zrows = jnp.zeros((K, D), jnp.float32)
    ones_rows = jnp.ones((K, D), jnp.float32)
    Wcat1 = jnp.concatenate([W_neigh1, W_self1], axis=1)
    Wcat2 = jnp.concatenate([W_neigh2, W_self2], axis=1)

    P1, S1 = _pre(n, Wcat1)
    (deg,) = _deg(ei, zrows, ones_rows)
    tok = (deg[0, 0] * 0.0).astype(jnp.int32)
    (agg1,) = _agg(P1, ei + tok, zrows)
    P2, S2 = _mid(S1, agg1[:NPAD], agg1[NPAD:], deg[:NPAD], deg[NPAD:],
                  b1.reshape(1, D), Wcat2)
    (agg2,) = _agg(P2, ei, zrows)
    y = _post(S2, agg2[:NPAD], agg2[NPAD:], deg[:NPAD], deg[NPAD:],
              b2.reshape(1, D),
              jnp.pad(W_aw, ((0, 0), (0, D - 1))),
              jnp.broadcast_to(b_aw.reshape(1, 1), (1, D)),
              W_t1, b_t1.reshape(1, D),
              jnp.pad(W_t2, ((0, 0), (0, D - 1))),
              jnp.broadcast_to(b_t2.reshape(1, 1), (1, D)))
    return y[:, 0:1]
